# Initial kernel scaffold; baseline (speedup 1.0000x reference)
import jax
import jax.numpy as jnp
from jax.experimental import pallas as pl


def kernel(x, log_w, tau_s):
    raise NotImplementedError("write your pallas kernel here")


# speedup vs baseline: 239.0952x; 239.0952x over previous
import functools

import jax
import jax.numpy as jnp
from jax import lax
from jax.experimental import pallas as pl
from jax.experimental.pallas import tpu as pltpu
from jax.experimental.pallas import tpu_sc as plsc

N = 16384
C = 512
ROW_W = 2 * C
TBL = 128
JT = 64
L = 16
NC, NS = 2, 16
NW = NC * NS
ROWS_PER_W = N // NW
CHUNK_ROWS = 32
NCHUNK = ROWS_PER_W // CHUNK_ROWS
CHUNK_ELEMS = CHUNK_ROWS * ROW_W


def _sc_body(x_hbm, lw_hbm, tau_hbm, out_hbm,
             lw_v, tau_v, f2_v, xb0, xb1, out_v, sem0, sem1):
    wid = lax.axis_index("s") * NC + lax.axis_index("c")
    row0 = wid * ROWS_PER_W

    pltpu.sync_copy(lw_hbm, lw_v)
    pltpu.sync_copy(tau_hbm, tau_v)

    bufs = (xb0, xb1)
    sems = (sem0, sem1)
    cps = {0: pltpu.async_copy(
        x_hbm.at[pl.ds(row0 * ROW_W, CHUNK_ELEMS)], xb0, sem0)}

    itau = 1.0 / tau_v[...]

    def f2body(kb, _):
        k = lax.iota(jnp.int32, L) + kb * L
        a = lax.shift_right_logical(k, 6)
        b = k & (JT - 1)
        d = (a - b) & (TBL - 1)
        lwv = plsc.load_gather(lw_v, [d])
        m = jnp.maximum(a, b).astype(jnp.float32)
        f2_v[pl.ds(kb * L, L)] = jnp.exp(lwv - (2.0 - m) * itau)
        return 0

    lax.fori_loop(0, (JT * JT) // L, f2body, 0)

    lane = lax.iota(jnp.int32, L)
    for ch in range(NCHUNK):
        nxt = ch + 1
        if nxt < NCHUNK:
            cps[nxt] = pltpu.async_copy(
                x_hbm.at[pl.ds((row0 + nxt * CHUNK_ROWS) * ROW_W,
                               CHUNK_ELEMS)],
                bufs[nxt % 2], sems[nxt % 2])
        cps[ch].wait()
        buf = bufs[ch % 2]
        for g in range(CHUNK_ROWS // L):
            base = lane * ROW_W + g * (L * ROW_W)

            def cbody(c, acc, base=base, buf=buf):
                i0 = base + c
                x0 = plsc.load_gather(buf, [i0])
                x1 = plsc.load_gather(buf, [i0 + C])
                t0 = ((1.0 - x0) * 63.0).astype(jnp.int32)
                t1 = ((1.0 - x1) * 63.0).astype(jnp.int32)
                key = t0 * JT + t1
                return acc + plsc.load_gather(f2_v, [key])

            acc = lax.fori_loop(0, C, cbody, jnp.zeros((L,), jnp.float32))
            out_v[pl.ds(ch * CHUNK_ROWS + g * L, L)] = acc

    pltpu.sync_copy(out_v, out_hbm.at[pl.ds(row0, ROWS_PER_W)])


@jax.jit
def kernel(x, log_w, tau_s):
    mesh = plsc.VectorSubcoreMesh(
        core_axis_name="c", subcore_axis_name="s",
        num_cores=NC, num_subcores=NS)
    run = pl.kernel(
        _sc_body,
        out_type=jax.ShapeDtypeStruct((N,), jnp.float32),
        mesh=mesh,
        scratch_types=[
            pltpu.VMEM((TBL,), jnp.float32),
            pltpu.VMEM((L,), jnp.float32),
            pltpu.VMEM((JT * JT,), jnp.float32),
            pltpu.VMEM((CHUNK_ELEMS,), jnp.float32),
            pltpu.VMEM((CHUNK_ELEMS,), jnp.float32),
            pltpu.VMEM((ROWS_PER_W,), jnp.float32),
            pltpu.SemaphoreType.DMA,
            pltpu.SemaphoreType.DMA,
        ],
        compiler_params=pltpu.CompilerParams(needs_layout_passes=False),
    )
    xf = x.reshape(N * ROW_W)
    tau16 = jnp.broadcast_to(tau_s.astype(jnp.float32), (L,))
    out = run(xf, log_w.astype(jnp.float32), tau16)
    return out.reshape(N, 1)


# speedup vs baseline: 252.2821x; 1.0552x over previous
import functools

import jax
import jax.numpy as jnp
from jax import lax
from jax.experimental import pallas as pl
from jax.experimental.pallas import tpu as pltpu
from jax.experimental.pallas import tpu_sc as plsc

N = 16384
C = 512
ROW_W = 2 * C
TBL = 128
JT = 64
L = 16
NC, NS = 2, 16
NW = NC * NS
ROWS_PER_W = N // NW
CHUNK_ROWS = 32
NCHUNK = ROWS_PER_W // CHUNK_ROWS
CHUNK_ELEMS = CHUNK_ROWS * ROW_W
UNROLL = 4


def _sc_body(x_hbm, lw_hbm, tau_hbm, out_hbm,
             lw_v, tau_v, f2_v, xb0, xb1, out_v, sem0, sem1):
    wid = lax.axis_index("s") * NC + lax.axis_index("c")
    row0 = wid * ROWS_PER_W

    pltpu.sync_copy(lw_hbm, lw_v)
    pltpu.sync_copy(tau_hbm, tau_v)

    bufs = (xb0, xb1)
    sems = (sem0, sem1)
    cps = {0: pltpu.async_copy(
        x_hbm.at[pl.ds(row0 * ROW_W, CHUNK_ELEMS)], xb0, sem0)}

    itau = 1.0 / tau_v[...]

    def f2body(kb, _):
        k = lax.iota(jnp.int32, L) + kb * L
        a = lax.shift_right_logical(k, 6)
        b = k & (JT - 1)
        d = (a - b) & (TBL - 1)
        lwv = plsc.load_gather(lw_v, [d])
        m = jnp.maximum(a, b).astype(jnp.float32)
        f2_v[pl.ds(kb * L, L)] = jnp.exp(lwv - (2.0 - m) * itau)
        return 0

    lax.fori_loop(0, (JT * JT) // L, f2body, 0)

    lane = lax.iota(jnp.int32, L)
    for ch in range(NCHUNK):
        nxt = ch + 1
        if nxt < NCHUNK:
            cps[nxt] = pltpu.async_copy(
                x_hbm.at[pl.ds((row0 + nxt * CHUNK_ROWS) * ROW_W,
                               CHUNK_ELEMS)],
                bufs[nxt % 2], sems[nxt % 2])
        cps[ch].wait()
        buf = bufs[ch % 2]
        for g in range(CHUNK_ROWS // L):
            base0 = lane * ROW_W + g * (L * ROW_W)
            base1 = base0 + C

            def cbody(c, acc, base0=base0, base1=base1, buf=buf):
                vals = []
                for u in range(UNROLL):
                    x0 = plsc.load_gather(buf, [base0 + (c + u)])
                    x1 = plsc.load_gather(buf, [base1 + (c + u)])
                    t0 = ((1.0 - x0) * 63.0).astype(jnp.int32)
                    t1 = ((1.0 - x1) * 63.0).astype(jnp.int32)
                    vals.append(plsc.load_gather(f2_v, [t0 * JT + t1]))
                s = (vals[0] + vals[1]) + (vals[2] + vals[3])
                return acc + s

            acc = plsc.parallel_loop(
                0, C, step=UNROLL, unroll=2,
                carry=jnp.zeros((L,), jnp.float32))(cbody)
            out_v[pl.ds(ch * CHUNK_ROWS + g * L, L)] = acc

    pltpu.sync_copy(out_v, out_hbm.at[pl.ds(row0, ROWS_PER_W)])


@jax.jit
def kernel(x, log_w, tau_s):
    mesh = plsc.VectorSubcoreMesh(
        core_axis_name="c", subcore_axis_name="s",
        num_cores=NC, num_subcores=NS)
    run = pl.kernel(
        _sc_body,
        out_type=jax.ShapeDtypeStruct((N,), jnp.float32),
        mesh=mesh,
        scratch_types=[
            pltpu.VMEM((TBL,), jnp.float32),
            pltpu.VMEM((L,), jnp.float32),
            pltpu.VMEM((JT * JT,), jnp.float32),
            pltpu.VMEM((CHUNK_ELEMS,), jnp.float32),
            pltpu.VMEM((CHUNK_ELEMS,), jnp.float32),
            pltpu.VMEM((ROWS_PER_W,), jnp.float32),
            pltpu.SemaphoreType.DMA,
            pltpu.SemaphoreType.DMA,
        ],
        compiler_params=pltpu.CompilerParams(needs_layout_passes=False),
    )
    xf = x.reshape(N * ROW_W)
    tau16 = jnp.broadcast_to(tau_s.astype(jnp.float32), (L,))
    out = run(xf, log_w.astype(jnp.float32), tau16)
    return out.reshape(N, 1)


# speedup vs baseline: 290.6799x; 1.1522x over previous
import functools

import jax
import jax.numpy as jnp
from jax import lax
from jax.experimental import pallas as pl
from jax.experimental.pallas import tpu as pltpu
from jax.experimental.pallas import tpu_sc as plsc

N = 16384
C = 512
ROW_W = 2 * C
TBL = 128
JT = 64
L = 16
NC, NS = 2, 16
NW = NC * NS
ROWS_PER_W = N // NW
CHUNK_ROWS = 32
NCHUNK = ROWS_PER_W // CHUNK_ROWS
CHUNK_ELEMS = CHUNK_ROWS * ROW_W
UNROLL = 4


def _sc_body(x_hbm, lw_hbm, tau_hbm, out_hbm,
             lw_v, tau_v, f2_v, xb0, xb1, out_v, sem0, sem1):
    wid = lax.axis_index("s") * NC + lax.axis_index("c")
    row0 = wid * ROWS_PER_W

    pltpu.sync_copy(lw_hbm, lw_v)
    pltpu.sync_copy(tau_hbm, tau_v)

    bufs = (xb0, xb1)
    sems = (sem0, sem1)
    cps = {0: pltpu.async_copy(
        x_hbm.at[pl.ds(row0, CHUNK_ROWS)], xb0, sem0)}

    itau = 1.0 / tau_v[...]

    def f2body(kb, _):
        k = lax.iota(jnp.int32, L) + kb * L
        a = lax.shift_right_logical(k, 6)
        b = k & (JT - 1)
        d = (a - b) & (TBL - 1)
        lwv = plsc.load_gather(lw_v, [d])
        m = jnp.maximum(a, b).astype(jnp.float32)
        f2_v[pl.ds(kb * L, L)] = jnp.exp(lwv - (2.0 - m) * itau)
        return 0

    lax.fori_loop(0, (JT * JT) // L, f2body, 0)

    lane = lax.iota(jnp.int32, L)
    for ch in range(NCHUNK):
        nxt = ch + 1
        if nxt < NCHUNK:
            cps[nxt] = pltpu.async_copy(
                x_hbm.at[pl.ds(row0 + nxt * CHUNK_ROWS, CHUNK_ROWS)],
                bufs[nxt % 2], sems[nxt % 2])
        cps[ch].wait()
        buf = bufs[ch % 2]
        zero16 = jnp.zeros((L,), jnp.int32)
        one16 = zero16 + 1
        for g in range(CHUNK_ROWS // L):
            rbase = lane + g * L

            def cbody(c, acc, rbase=rbase, buf=buf):
                vals = []
                for u in range(UNROLL):
                    cv = zero16 + (c + u)
                    x0 = plsc.load_gather(buf, [rbase, zero16, cv])
                    x1 = plsc.load_gather(buf, [rbase, one16, cv])
                    t0 = ((1.0 - x0) * 63.0).astype(jnp.int32)
                    t1 = ((1.0 - x1) * 63.0).astype(jnp.int32)
                    vals.append(plsc.load_gather(f2_v, [t0 * JT + t1]))
                s = (vals[0] + vals[1]) + (vals[2] + vals[3])
                return acc + s

            acc = plsc.parallel_loop(
                0, C, step=UNROLL, unroll=2,
                carry=jnp.zeros((L,), jnp.float32))(cbody)
            out_v[pl.ds(ch * CHUNK_ROWS + g * L, L)] = acc

    pltpu.sync_copy(out_v, out_hbm.at[pl.ds(row0, ROWS_PER_W)])


@jax.jit
def kernel(x, log_w, tau_s):
    mesh = plsc.VectorSubcoreMesh(
        core_axis_name="c", subcore_axis_name="s",
        num_cores=NC, num_subcores=NS)
    run = pl.kernel(
        _sc_body,
        out_type=jax.ShapeDtypeStruct((N,), jnp.float32),
        mesh=mesh,
        scratch_types=[
            pltpu.VMEM((TBL,), jnp.float32),
            pltpu.VMEM((L,), jnp.float32),
            pltpu.VMEM((JT * JT,), jnp.float32),
            pltpu.VMEM((CHUNK_ROWS, 2, C), jnp.float32),
            pltpu.VMEM((CHUNK_ROWS, 2, C), jnp.float32),
            pltpu.VMEM((ROWS_PER_W,), jnp.float32),
            pltpu.SemaphoreType.DMA,
            pltpu.SemaphoreType.DMA,
        ],
        compiler_params=pltpu.CompilerParams(needs_layout_passes=False),
    )
    tau16 = jnp.broadcast_to(tau_s.astype(jnp.float32), (L,))
    out = run(x, log_w.astype(jnp.float32), tau16)
    return out.reshape(N, 1)


# speedup vs baseline: 948.4820x; 3.2630x over previous
import functools

import jax
import jax.numpy as jnp
from jax import lax
from jax.experimental import pallas as pl
from jax.experimental.pallas import tpu as pltpu
from jax.experimental.pallas import tpu_sc as plsc

N = 16384
C = 512
ROW_W = 2 * C
TBL = 128
JT = 64
L = 16
NC, NS = 2, 16
NW = NC * NS
ROWS_PER_W = N // NW
CHUNK_ROWS = 32
NCHUNK = ROWS_PER_W // CHUNK_ROWS
CHUNK_ELEMS = CHUNK_ROWS * ROW_W
UNROLL = 4


def _sc_body(x_hbm, lw_hbm, tau_hbm, out_hbm,
             lw_v, tau_v, f2_v, xb0, xb1, out_v, sem0, sem1):
    wid = lax.axis_index("s") * NC + lax.axis_index("c")
    row0 = wid * ROWS_PER_W

    pltpu.sync_copy(lw_hbm, lw_v)
    pltpu.sync_copy(tau_hbm, tau_v)

    bufs = (xb0, xb1)
    sems = (sem0, sem1)
    cps = {0: pltpu.async_copy(
        x_hbm.at[pl.ds(row0, CHUNK_ROWS)], xb0, sem0)}

    itau = 1.0 / tau_v[...]

    def f2body(kb, _):
        k = lax.iota(jnp.int32, L) + kb * L
        a = lax.shift_right_logical(k, 6)
        b = k & (JT - 1)
        d = (a - b) & (TBL - 1)
        lwv = plsc.load_gather(lw_v, [d])
        m = jnp.maximum(a, b).astype(jnp.float32)
        f2_v[pl.ds(kb * L, L)] = jnp.exp(lwv - (2.0 - m) * itau)
        return 0

    lax.fori_loop(0, (JT * JT) // L, f2body, 0)

    lane = lax.iota(jnp.int32, L)
    lane0 = lane == 0
    zero16 = jnp.zeros((L,), jnp.int32)
    for ch in range(NCHUNK):
        nxt = ch + 1
        if nxt < NCHUNK:
            cps[nxt] = pltpu.async_copy(
                x_hbm.at[pl.ds(row0 + nxt * CHUNK_ROWS, CHUNK_ROWS)],
                bufs[nxt % 2], sems[nxt % 2])
        cps[ch].wait()
        buf = bufs[ch % 2]

        def row_body(rr, _, buf=buf, ch=ch):
            def cbody(c, acc, rr=rr, buf=buf):
                x0 = buf[rr, 0, pl.ds(c, L)]
                x1 = buf[rr, 1, pl.ds(c, L)]
                t0 = ((1.0 - x0) * 63.0).astype(jnp.int32)
                t1 = ((1.0 - x1) * 63.0).astype(jnp.int32)
                return acc + plsc.load_gather(f2_v, [t0 * JT + t1])

            acc = plsc.parallel_loop(
                0, C, step=L, unroll=UNROLL,
                carry=jnp.zeros((L,), jnp.float32))(cbody)
            s = jnp.sum(acc)
            plsc.store_scatter(
                out_v, [zero16 + (ch * CHUNK_ROWS + rr)],
                jnp.zeros((L,), jnp.float32) + s, mask=lane0)
            return 0

        lax.fori_loop(0, CHUNK_ROWS, row_body, 0)

    pltpu.sync_copy(out_v, out_hbm.at[pl.ds(row0, ROWS_PER_W)])


@jax.jit
def kernel(x, log_w, tau_s):
    mesh = plsc.VectorSubcoreMesh(
        core_axis_name="c", subcore_axis_name="s",
        num_cores=NC, num_subcores=NS)
    run = pl.kernel(
        _sc_body,
        out_type=jax.ShapeDtypeStruct((N,), jnp.float32),
        mesh=mesh,
        scratch_types=[
            pltpu.VMEM((TBL,), jnp.float32),
            pltpu.VMEM((L,), jnp.float32),
            pltpu.VMEM((JT * JT,), jnp.float32),
            pltpu.VMEM((CHUNK_ROWS, 2, C), jnp.float32),
            pltpu.VMEM((CHUNK_ROWS, 2, C), jnp.float32),
            pltpu.VMEM((ROWS_PER_W,), jnp.float32),
            pltpu.SemaphoreType.DMA,
            pltpu.SemaphoreType.DMA,
        ],
        compiler_params=pltpu.CompilerParams(needs_layout_passes=False),
    )
    tau16 = jnp.broadcast_to(tau_s.astype(jnp.float32), (L,))
    out = run(x, log_w.astype(jnp.float32), tau16)
    return out.reshape(N, 1)
